# Initial kernel scaffold; baseline (speedup 1.0000x reference)
import jax
import jax.numpy as jnp
from jax.experimental import pallas as pl


def kernel(X_target, X_context1, X_context2, W1, b1, W2, b2):
    raise NotImplementedError("write your pallas kernel here")


# speedup vs baseline: 4.5746x; 4.5746x over previous
import jax
import jax.numpy as jnp
from jax import lax
from jax.experimental import pallas as pl

F_DIM = 128
THRESH = 0.1
B = 2
N = 1024
N_CTX = 2048
S = N + N_CTX
V = B * S


def _hyper_kernel(x_ref, w1_ref, b1_ref, w2_ref, b2_ref, out_ref):
    x = x_ref[:]
    w1 = w1_ref[:]
    w2 = w2_ref[:]
    b1 = b1_ref[:]
    b2 = b2_ref[:]

    nrm = jnp.sqrt(jnp.sum(x * x, axis=1, keepdims=True))
    xhat = x / jnp.maximum(nrm, 1e-8)

    xn = jnp.dot(x, w1, preferred_element_type=jnp.float32) + b1

    for i in range(B):
        xt_n = xhat[i * N:(i + 1) * N]
        xc_n = xhat[B * N + i * N_CTX:B * N + (i + 1) * N_CTX]
        sim = lax.dot_general(
            xt_n, xc_n, (((1,), (1,)), ((), ())),
            preferred_element_type=jnp.float32)
        m = (sim > THRESH).astype(jnp.float32)

        y = xn[i * S:(i + 1) * S]
        deg_e = 1.0 + jnp.sum(m, axis=1, keepdims=True)
        xe = (y[:N] + jnp.dot(m, y[N:], preferred_element_type=jnp.float32)) / deg_e
        xet = jnp.dot(xe, w2, preferred_element_type=jnp.float32) + b2

        out_ref[i * S:i * S + N, :] = xet
        deg_v = jnp.maximum(jnp.sum(m, axis=0, keepdims=True), 1.0)
        bot = lax.dot_general(
            m, xet, (((0,), (0,)), ((), ())),
            preferred_element_type=jnp.float32)
        out_ref[i * S + N:(i + 1) * S, :] = bot / deg_v.T


def kernel(X_target, X_context1, X_context2, W1, b1, W2, b2):
    Bb, C, Hh, Ww = X_target.shape
    n = Hh * Ww
    to_rows = lambda a: jnp.transpose(a, (0, 2, 3, 1)).reshape(Bb * n, C)
    Xt = to_rows(X_target)
    Xc1 = jnp.transpose(X_context1, (0, 2, 3, 1)).reshape(Bb, n, C)
    Xc2 = jnp.transpose(X_context2, (0, 2, 3, 1)).reshape(Bb, n, C)
    Xc = jnp.concatenate([Xc1, Xc2], axis=1).reshape(Bb * 2 * n, C)
    x_all = jnp.concatenate([Xt, Xc], axis=0)

    x_new = pl.pallas_call(
        _hyper_kernel,
        out_shape=jax.ShapeDtypeStruct((V, F_DIM), jnp.float32),
    )(x_all, W1, b1.reshape(1, F_DIM), W2, b2.reshape(1, F_DIM))

    to_nchw = lambda a: jnp.transpose(a, (0, 3, 1, 2))
    xt_out = to_nchw(x_new[:Bb * n].reshape(Bb, Hh, Ww, C))
    xc_out = x_new[Bb * n:].reshape(Bb, 2 * n, C)
    xc1_out = to_nchw(xc_out[:, :n, :].reshape(Bb, Hh, Ww, C))
    xc2_out = to_nchw(xc_out[:, n:, :].reshape(Bb, Hh, Ww, C))
    return (xt_out, xc1_out, xc2_out)
